# baseline (speedup 1.0000x reference)
import math

import jax
import jax.numpy as jnp
from jax.experimental import pallas as pl

B = 64
S = 8192
OUTPUT_DIM = 64
NUM_BINS = 32
MAX_DISTANCE = 1e7
HALF = OUTPUT_DIM // 2

_INV_SQRT2 = 0.7071067811865476
_RB = 8


def _encoder_kernel(d_ref, edges_ref, w1_ref, b1_ref, wf_ref, bias_ref, out_ref):
    r = pl.program_id(1)
    d = d_ref[pl.ds(r, 1), :]
    cmp_t = (d > edges_ref[...]).astype(jnp.float32)
    ld = jnp.log1p(d * 1e-3)
    pre = ld * w1_ref[...] + b1_ref[...]
    h_t = 0.5 * pre * (1.0 + jax.lax.erf(pre * _INV_SQRT2))
    xt = jnp.concatenate([cmp_t, h_t], axis=0)
    out_ref[...] = (
        jax.lax.dot_general(
            xt, wf_ref[...],
            dimension_numbers=(((0,), (0,)), ((), ())),
            preferred_element_type=jnp.float32,
        )
        + bias_ref[...]
    )


def kernel(distances, emb, W1, b1, W2, b2, Wc, bc):
    N = B * S

    edges = jnp.logspace(3.0, math.log10(MAX_DISTANCE), NUM_BINS,
                         dtype=jnp.float32)
    Temb = emb @ Wc[:OUTPUT_DIM]
    dT = Temb[1:] - Temb[:-1]
    Wh = W2 @ Wc[OUTPUT_DIM:]
    Wf = jnp.concatenate([dT, Wh], axis=0)
    bias = (Temb[0] + b2 @ Wc[OUTPUT_DIM:] + bc).reshape(1, OUTPUT_DIM)

    edges_col = edges.reshape(NUM_BINS, 1)
    w1_col = W1.reshape(HALF, 1)
    b1_col = b1.reshape(HALF, 1)

    CH = 4
    BC = B // CH
    parts = []
    for k in range(CH):
        part = pl.pallas_call(
            _encoder_kernel,
            grid=(BC // _RB, _RB),
            in_specs=[
                pl.BlockSpec((_RB, S), lambda i, j: (i, 0)),
                pl.BlockSpec((NUM_BINS, 1), lambda i, j: (0, 0)),
                pl.BlockSpec((HALF, 1), lambda i, j: (0, 0)),
                pl.BlockSpec((HALF, 1), lambda i, j: (0, 0)),
                pl.BlockSpec((OUTPUT_DIM, OUTPUT_DIM), lambda i, j: (0, 0)),
                pl.BlockSpec((1, OUTPUT_DIM), lambda i, j: (0, 0)),
            ],
            out_specs=pl.BlockSpec((S, OUTPUT_DIM),
                                   lambda i, j: (i * _RB + j, 0)),
            out_shape=jax.ShapeDtypeStruct((BC * S, OUTPUT_DIM), jnp.float32),
        )(distances[k * BC:(k + 1) * BC], edges_col, w1_col, b1_col, Wf, bias)
        parts.append(part.reshape(BC, S, OUTPUT_DIM))

    return jnp.concatenate(parts, axis=0)


# speedup vs baseline: 1.3841x; 1.3841x over previous
import math

import jax
import jax.numpy as jnp
from jax.experimental import pallas as pl

B = 64
S = 8192
OUTPUT_DIM = 64
NUM_BINS = 32
MAX_DISTANCE = 1e7
HALF = OUTPUT_DIM // 2

_INV_SQRT2 = 0.7071067811865476
_RB = 8


def _encoder_kernel(d_ref, edges_ref, w1_ref, b1_ref, wf_ref, bias_ref, out_ref):
    r = pl.program_id(1)
    d = d_ref[pl.ds(r, 1), :]
    cmp_t = (d > edges_ref[...]).astype(jnp.float32)
    ld = jnp.log1p(d * 1e-3)
    pre = ld * w1_ref[...] + b1_ref[...]
    h_t = 0.5 * pre * (1.0 + jax.lax.erf(pre * _INV_SQRT2))
    xt = jnp.concatenate([cmp_t, h_t], axis=0)
    out_ref[...] = (
        jax.lax.dot_general(
            xt, wf_ref[...],
            dimension_numbers=(((0,), (0,)), ((), ())),
            preferred_element_type=jnp.float32,
        )
        + bias_ref[...]
    )


def kernel(distances, emb, W1, b1, W2, b2, Wc, bc):
    N = B * S

    edges = jnp.logspace(3.0, math.log10(MAX_DISTANCE), NUM_BINS,
                         dtype=jnp.float32)
    Temb = emb @ Wc[:OUTPUT_DIM]
    dT = Temb[1:] - Temb[:-1]
    Wh = W2 @ Wc[OUTPUT_DIM:]
    Wf = jnp.concatenate([dT, Wh], axis=0)
    bias = (Temb[0] + b2 @ Wc[OUTPUT_DIM:] + bc).reshape(1, OUTPUT_DIM)

    edges_col = edges.reshape(NUM_BINS, 1)
    w1_col = W1.reshape(HALF, 1)
    b1_col = b1.reshape(HALF, 1)

    grid = (B // _RB, _RB)
    out = pl.pallas_call(
        _encoder_kernel,
        grid=grid,
        in_specs=[
            pl.BlockSpec((_RB, S), lambda i, j: (i, 0)),
            pl.BlockSpec((NUM_BINS, 1), lambda i, j: (0, 0)),
            pl.BlockSpec((HALF, 1), lambda i, j: (0, 0)),
            pl.BlockSpec((HALF, 1), lambda i, j: (0, 0)),
            pl.BlockSpec((OUTPUT_DIM, OUTPUT_DIM), lambda i, j: (0, 0)),
            pl.BlockSpec((1, OUTPUT_DIM), lambda i, j: (0, 0)),
        ],
        out_specs=pl.BlockSpec((S, OUTPUT_DIM), lambda i, j: (i * _RB + j, 0)),
        out_shape=jax.ShapeDtypeStruct((N, OUTPUT_DIM), jnp.float32),
    )(distances, edges_col, w1_col, b1_col, Wf, bias)

    return out.reshape(B, S, OUTPUT_DIM)
